# baseline (speedup 1.0000x reference)
import functools

import jax
import jax.numpy as jnp
from jax import lax
from jax.experimental import pallas as pl
from jax.experimental.pallas import tpu as pltpu
from jax.experimental.pallas import tpu_sc as plsc

NC = 2
NS = 16
NW = NC * NS

B = 100000
D = 128
PER_W = 3128
CHUNK = 128
FULL_CHUNKS = PER_W // CHUNK
TAIL = PER_W - FULL_CHUNKS * CHUNK
LAST_BASE = B - PER_W
RING = 7
DEPTH = 3


def _gather_body(idx_hbm, table_hbm, out_hbm, idx_v, rows_v, tail_v, gsem, wsem, tsem):
    wid = lax.axis_index("s") * NC + lax.axis_index("c")
    base = pl.multiple_of(jnp.minimum(wid * PER_W, LAST_BASE), 8)
    pltpu.sync_copy(idx_hbm.at[pl.ds(base, PER_W)], idx_v)
    tail = pltpu.async_copy(
        table_hbm.at[idx_v.at[pl.ds(FULL_CHUNKS * CHUNK, TAIL)]], tail_v, tsem)

    def start_gather(c):
        return pltpu.async_copy(
            table_hbm.at[idx_v.at[pl.ds(c * CHUNK, CHUNK)]],
            rows_v.at[c % RING], gsem.at[c % RING])

    gathers = {c: start_gather(c) for c in range(DEPTH)}
    writes = {}
    for j in range(FULL_CHUNKS):
        b = j % RING
        gathers[j].wait()
        writes[j] = pltpu.async_copy(
            rows_v.at[b], out_hbm.at[pl.ds(base + j * CHUNK, CHUNK)], wsem.at[b])
        c = j + DEPTH
        if c < FULL_CHUNKS:
            pw = c - RING
            if pw >= 0:
                writes[pw].wait()
            gathers[c] = start_gather(c)
    for j in range(FULL_CHUNKS - RING, FULL_CHUNKS):
        writes[j].wait()
    tail.wait()
    pltpu.sync_copy(tail_v, out_hbm.at[pl.ds(base + FULL_CHUNKS * CHUNK, TAIL)])


@jax.jit
def _gather(idx, table):
    mesh = plsc.VectorSubcoreMesh(core_axis_name="c", subcore_axis_name="s")
    run = functools.partial(
        pl.kernel,
        mesh=mesh,
        out_type=jax.ShapeDtypeStruct((B, D), jnp.float32),
        scratch_types=[
            pltpu.VMEM((PER_W,), jnp.int32),
            pltpu.VMEM((RING, CHUNK, D), jnp.float32),
            pltpu.VMEM((TAIL, D), jnp.float32),
            pltpu.SemaphoreType.DMA((RING,)),
            pltpu.SemaphoreType.DMA((RING,)),
            pltpu.SemaphoreType.DMA,
        ],
    )(_gather_body)
    return run(idx, table)


def kernel(f_nodes, f_edges, node2edge, edge2node, b2revb, edit_embedding):
    return _gather(f_nodes.astype(jnp.int32), edit_embedding)


# speedup vs baseline: 1.0268x; 1.0268x over previous
import functools

import jax
import jax.numpy as jnp
from jax import lax
from jax.experimental import pallas as pl
from jax.experimental.pallas import tpu as pltpu
from jax.experimental.pallas import tpu_sc as plsc

NC = 2
NS = 16
NW = NC * NS

B = 100000
D = 128
PER_W = 3128
CHUNK = 256
FULL_CHUNKS = PER_W // CHUNK
TAIL = PER_W - FULL_CHUNKS * CHUNK
LAST_BASE = B - PER_W
RING = 3


def _gather_body(idx_hbm, table_hbm, out_hbm, idx_v, rows_v, tail_v, gsem, tsem):
    wid = lax.axis_index("s") * NC + lax.axis_index("c")
    base = pl.multiple_of(jnp.minimum(wid * PER_W, LAST_BASE), 8)
    pltpu.sync_copy(idx_hbm.at[pl.ds(base, PER_W)], idx_v)
    tail = pltpu.async_copy(
        table_hbm.at[idx_v.at[pl.ds(FULL_CHUNKS * CHUNK, TAIL)]], tail_v, tsem)

    def start_gather(c):
        return pltpu.async_copy(
            table_hbm.at[idx_v.at[pl.ds(c * CHUNK, CHUNK)]],
            rows_v.at[c % RING], gsem.at[c % RING])

    gathers = {c: start_gather(c) for c in range(RING)}
    for j in range(FULL_CHUNKS):
        b = j % RING
        gathers[j].wait()
        pltpu.sync_copy(rows_v.at[b], out_hbm.at[pl.ds(base + j * CHUNK, CHUNK)])
        if j + RING < FULL_CHUNKS:
            gathers[j + RING] = start_gather(j + RING)
    tail.wait()
    pltpu.sync_copy(tail_v, out_hbm.at[pl.ds(base + FULL_CHUNKS * CHUNK, TAIL)])


@jax.jit
def _gather(idx, table):
    mesh = plsc.VectorSubcoreMesh(core_axis_name="c", subcore_axis_name="s")
    run = functools.partial(
        pl.kernel,
        mesh=mesh,
        out_type=jax.ShapeDtypeStruct((B, D), jnp.float32),
        scratch_types=[
            pltpu.VMEM((PER_W,), jnp.int32),
            pltpu.VMEM((RING, CHUNK, D), jnp.float32),
            pltpu.VMEM((TAIL, D), jnp.float32),
            pltpu.SemaphoreType.DMA((RING,)),
            pltpu.SemaphoreType.DMA,
        ],
    )(_gather_body)
    return run(idx, table)


def kernel(f_nodes, f_edges, node2edge, edge2node, b2revb, edit_embedding):
    return _gather(f_nodes.astype(jnp.int32), edit_embedding)
